# initial kernel scaffold (unmeasured)
import jax
import jax.numpy as jnp
from jax import lax
from jax.experimental import pallas as pl
from jax.experimental.pallas import tpu as pltpu

B, S, D = 2, 512, 2048
H, Dh, Dr = 16, 128, 32
DC = 128
BS = B * S
DP = 256
SCALE = (Dh + Dr) ** -0.5


def _kv_exchange_body(x_ref, wdkv_ref, wuk_ref, wuv_ref, wkr_ref,
                      kbig_ref, v_ref,
                      c_send, c_recv, wuk_recv, wuv_recv,
                      c_full, wuk_full, wuv_full,
                      send_sems, recv_sems):
    my_x = lax.axis_index("x")
    my_y = lax.axis_index("y")
    my_z = lax.axis_index("z")
    peer = (my_x, my_y, 1 - my_z)

    barrier_sem = pltpu.get_barrier_semaphore()
    pl.semaphore_signal(barrier_sem, inc=1, device_id=peer,
                        device_id_type=pl.DeviceIdType.MESH)
    pl.semaphore_wait(barrier_sem, 1)

    rdma_wuk = pltpu.make_async_remote_copy(
        src_ref=wuk_ref, dst_ref=wuk_recv,
        send_sem=send_sems.at[0], recv_sem=recv_sems.at[0],
        device_id=peer, device_id_type=pl.DeviceIdType.MESH)
    rdma_wuk.start()
    rdma_wuv = pltpu.make_async_remote_copy(
        src_ref=wuv_ref, dst_ref=wuv_recv,
        send_sem=send_sems.at[1], recv_sem=recv_sems.at[1],
        device_id=peer, device_id_type=pl.DeviceIdType.MESH)
    rdma_wuv.start()

    c_send[...] = jnp.dot(x_ref[...], wdkv_ref[...],
                          preferred_element_type=jnp.float32)
    rdma_c = pltpu.make_async_remote_copy(
        src_ref=c_send, dst_ref=c_recv,
        send_sem=send_sems.at[2], recv_sem=recv_sems.at[2],
        device_id=peer, device_id_type=pl.DeviceIdType.MESH)
    rdma_c.start()

    kr = jnp.dot(x_ref[...], wkr_ref[...],
                 preferred_element_type=jnp.float32)

    rdma_wuk.wait()
    rdma_wuv.wait()
    rdma_c.wait()

    c_full[:, pl.ds(my_z * DC, DC)] = c_send[...]
    c_full[:, pl.ds((1 - my_z) * DC, DC)] = c_recv[...]
    wuk_full[pl.ds(my_z * DC, DC), :] = wuk_ref[...]
    wuk_full[pl.ds((1 - my_z) * DC, DC), :] = wuk_recv[...]
    wuv_full[pl.ds(my_z * DC, DC), :] = wuv_ref[...]
    wuv_full[pl.ds((1 - my_z) * DC, DC), :] = wuv_recv[...]

    k2d = jnp.dot(c_full[...], wuk_full[...],
                  preferred_element_type=jnp.float32)
    v2d = jnp.dot(c_full[...], wuv_full[...],
                  preferred_element_type=jnp.float32)

    kbig_ref[:, :, 0:Dh] = k2d.reshape(BS, H, Dh)
    kbig_ref[:, :, Dh:Dh + Dr] = jnp.broadcast_to(kr[:, None, :], (BS, H, Dr))
    kbig_ref[:, :, Dh + Dr:] = jnp.zeros((BS, H, DP - Dh - Dr), jnp.float32)
    v_ref[...] = v2d.reshape(BS, H, Dh)


def _q_proj_body(x_ref, wq_ref, wqr_ref, qbig_ref):
    q2d = jnp.dot(x_ref[...], wq_ref[...], preferred_element_type=jnp.float32)
    qr2d = jnp.dot(x_ref[...], wqr_ref[...], preferred_element_type=jnp.float32)
    qbig_ref[:, :, 0:Dh] = q2d.reshape(BS, H, Dh)
    qbig_ref[:, :, Dh:Dh + Dr] = qr2d.reshape(BS, H, Dr)
    qbig_ref[:, :, Dh + Dr:] = jnp.zeros((BS, H, DP - Dh - Dr), jnp.float32)


def _attn_body(q_ref, k_ref, v_ref, o_ref):
    q = q_ref[:, 0, :]
    k = k_ref[:, 0, :]
    v = v_ref[:, 0, :]
    scores = lax.dot_general(q, k, (((1,), (1,)), ((), ())),
                             preferred_element_type=jnp.float32) * SCALE
    m = jnp.max(scores, axis=1, keepdims=True)
    p = jnp.exp(scores - m)
    p = p / jnp.sum(p, axis=1, keepdims=True)
    o_ref[:, 0, :] = jnp.dot(p, v, preferred_element_type=jnp.float32)


def _out_proj_body(o_ref, wo_ref, out_ref):
    out_ref[...] = jnp.dot(o_ref[...], wo_ref[...],
                           preferred_element_type=jnp.float32)


def kernel(x, Wdkv, Wuk, Wuv, Wq, Wqr, Wkr, Wo):
    x2d = x.reshape(BS, D)

    vmem = pl.BlockSpec(memory_space=pltpu.VMEM)

    kbig, v = pl.pallas_call(
        _kv_exchange_body,
        out_shape=(
            jax.ShapeDtypeStruct((BS, H, DP), jnp.float32),
            jax.ShapeDtypeStruct((BS, H, Dh), jnp.float32),
        ),
        in_specs=[vmem] * 5,
        out_specs=(vmem, vmem),
        scratch_shapes=[
            pltpu.VMEM((BS, DC), jnp.float32),
            pltpu.VMEM((BS, DC), jnp.float32),
            pltpu.VMEM((DC, D), jnp.float32),
            pltpu.VMEM((DC, D), jnp.float32),
            pltpu.VMEM((BS, 2 * DC), jnp.float32),
            pltpu.VMEM((2 * DC, D), jnp.float32),
            pltpu.VMEM((2 * DC, D), jnp.float32),
            pltpu.SemaphoreType.DMA((3,)),
            pltpu.SemaphoreType.DMA((3,)),
        ],
        compiler_params=pltpu.CompilerParams(collective_id=0),
    )(x2d, Wdkv, Wuk, Wuv, Wkr)

    qbig = pl.pallas_call(
        _q_proj_body,
        out_shape=jax.ShapeDtypeStruct((BS, H, DP), jnp.float32),
        in_specs=[vmem] * 3,
        out_specs=vmem,
    )(x2d, Wq, Wqr)

    o = pl.pallas_call(
        _attn_body,
        grid=(B, H),
        in_specs=[
            pl.BlockSpec((S, 1, DP), lambda b, h: (b, h, 0)),
            pl.BlockSpec((S, 1, DP), lambda b, h: (b, h, 0)),
            pl.BlockSpec((S, 1, Dh), lambda b, h: (b, h, 0)),
        ],
        out_specs=pl.BlockSpec((S, 1, Dh), lambda b, h: (b, h, 0)),
        out_shape=jax.ShapeDtypeStruct((BS, H, Dh), jnp.float32),
    )(qbig, kbig, v)

    out2d = pl.pallas_call(
        _out_proj_body,
        out_shape=jax.ShapeDtypeStruct((BS, D), jnp.float32),
        in_specs=[vmem] * 2,
        out_specs=vmem,
    )(o.reshape(BS, D), Wo)

    return out2d.reshape(B, S, D)


# baseline (device time: 169486 ns/iter reference)
import jax
import jax.numpy as jnp
from jax import lax
from jax.experimental import pallas as pl
from jax.experimental.pallas import tpu as pltpu

B, S, D = 2, 512, 2048
H, Dh, Dr = 16, 128, 32
DC = 128
BS = B * S
DP = 256
SCALE = (Dh + Dr) ** -0.5
VMEM_LIMIT = 120 * 1024 * 1024


def _kv_exchange_body(x_ref, wdkv_ref, wuk_ref, wuv_ref, wkr_ref,
                      kbig_ref, v_ref,
                      c_send, c_recv, wuk_recv, wuv_recv,
                      c_full, wuk_full, wuv_full,
                      send_sems, recv_sems):
    my_x = lax.axis_index("x")
    my_y = lax.axis_index("y")
    my_z = lax.axis_index("z")
    peer = (my_x, my_y, 1 - my_z)

    barrier_sem = pltpu.get_barrier_semaphore()
    pl.semaphore_signal(barrier_sem, inc=1, device_id=peer,
                        device_id_type=pl.DeviceIdType.MESH)
    pl.semaphore_wait(barrier_sem, 1)

    rdma_wuk = pltpu.make_async_remote_copy(
        src_ref=wuk_ref, dst_ref=wuk_recv,
        send_sem=send_sems.at[0], recv_sem=recv_sems.at[0],
        device_id=peer, device_id_type=pl.DeviceIdType.MESH)
    rdma_wuk.start()
    rdma_wuv = pltpu.make_async_remote_copy(
        src_ref=wuv_ref, dst_ref=wuv_recv,
        send_sem=send_sems.at[1], recv_sem=recv_sems.at[1],
        device_id=peer, device_id_type=pl.DeviceIdType.MESH)
    rdma_wuv.start()

    c_send[...] = jnp.dot(x_ref[...], wdkv_ref[...],
                          preferred_element_type=jnp.float32)
    rdma_c = pltpu.make_async_remote_copy(
        src_ref=c_send, dst_ref=c_recv,
        send_sem=send_sems.at[2], recv_sem=recv_sems.at[2],
        device_id=peer, device_id_type=pl.DeviceIdType.MESH)
    rdma_c.start()

    kr = jnp.dot(x_ref[...], wkr_ref[...],
                 preferred_element_type=jnp.float32)

    rdma_wuk.wait()
    rdma_wuv.wait()
    rdma_c.wait()

    c_full[:, pl.ds(my_z * DC, DC)] = c_send[...]
    c_full[:, pl.ds((1 - my_z) * DC, DC)] = c_recv[...]
    wuk_full[pl.ds(my_z * DC, DC), :] = wuk_ref[...]
    wuk_full[pl.ds((1 - my_z) * DC, DC), :] = wuk_recv[...]
    wuv_full[pl.ds(my_z * DC, DC), :] = wuv_ref[...]
    wuv_full[pl.ds((1 - my_z) * DC, DC), :] = wuv_recv[...]

    k2d = jnp.dot(c_full[...], wuk_full[...],
                  preferred_element_type=jnp.float32)
    v2d = jnp.dot(c_full[...], wuv_full[...],
                  preferred_element_type=jnp.float32)

    kbig_ref[:, :, 0:Dh] = k2d.reshape(BS, H, Dh)
    kbig_ref[:, :, Dh:Dh + Dr] = jnp.broadcast_to(kr[:, None, :], (BS, H, Dr))
    kbig_ref[:, :, Dh + Dr:] = jnp.zeros((BS, H, DP - Dh - Dr), jnp.float32)
    v_ref[...] = v2d.reshape(BS, H, Dh)


def _q_proj_body(x_ref, wq_ref, wqr_ref, qbig_ref):
    q2d = jnp.dot(x_ref[...], wq_ref[...], preferred_element_type=jnp.float32)
    qr2d = jnp.dot(x_ref[...], wqr_ref[...], preferred_element_type=jnp.float32)
    qbig_ref[:, :, 0:Dh] = q2d.reshape(BS, H, Dh)
    qbig_ref[:, :, Dh:Dh + Dr] = qr2d.reshape(BS, H, Dr)
    qbig_ref[:, :, Dh + Dr:] = jnp.zeros((BS, H, DP - Dh - Dr), jnp.float32)


def _attn_body(q_ref, k_ref, v_ref, o_ref):
    for h in range(H):
        q = q_ref[:, h, :]
        k = k_ref[:, h, :]
        v = v_ref[:, h, :]
        scores = lax.dot_general(q, k, (((1,), (1,)), ((), ())),
                                 preferred_element_type=jnp.float32) * SCALE
        m = jnp.max(scores, axis=1, keepdims=True)
        p = jnp.exp(scores - m)
        p = p / jnp.sum(p, axis=1, keepdims=True)
        o_ref[:, h, :] = jnp.dot(p, v, preferred_element_type=jnp.float32)


def _out_proj_body(o_ref, wo_ref, out_ref):
    out_ref[...] = jnp.dot(o_ref[...], wo_ref[...],
                           preferred_element_type=jnp.float32)


def kernel(x, Wdkv, Wuk, Wuv, Wq, Wqr, Wkr, Wo):
    x2d = x.reshape(BS, D)

    vmem = pl.BlockSpec(memory_space=pltpu.VMEM)

    kbig, v = pl.pallas_call(
        _kv_exchange_body,
        out_shape=(
            jax.ShapeDtypeStruct((BS, H, DP), jnp.float32),
            jax.ShapeDtypeStruct((BS, H, Dh), jnp.float32),
        ),
        in_specs=[vmem] * 5,
        out_specs=(vmem, vmem),
        scratch_shapes=[
            pltpu.VMEM((BS, DC), jnp.float32),
            pltpu.VMEM((BS, DC), jnp.float32),
            pltpu.VMEM((DC, D), jnp.float32),
            pltpu.VMEM((DC, D), jnp.float32),
            pltpu.VMEM((BS, 2 * DC), jnp.float32),
            pltpu.VMEM((2 * DC, D), jnp.float32),
            pltpu.VMEM((2 * DC, D), jnp.float32),
            pltpu.SemaphoreType.DMA((3,)),
            pltpu.SemaphoreType.DMA((3,)),
        ],
        compiler_params=pltpu.CompilerParams(collective_id=0,
                                             vmem_limit_bytes=VMEM_LIMIT),
    )(x2d, Wdkv, Wuk, Wuv, Wkr)

    qbig = pl.pallas_call(
        _q_proj_body,
        out_shape=jax.ShapeDtypeStruct((BS, H, DP), jnp.float32),
        in_specs=[vmem] * 3,
        out_specs=vmem,
        compiler_params=pltpu.CompilerParams(vmem_limit_bytes=VMEM_LIMIT),
    )(x2d, Wq, Wqr)

    o = pl.pallas_call(
        _attn_body,
        grid=(B,),
        in_specs=[
            pl.BlockSpec((S, H, DP), lambda b: (b, 0, 0)),
            pl.BlockSpec((S, H, DP), lambda b: (b, 0, 0)),
            pl.BlockSpec((S, H, Dh), lambda b: (b, 0, 0)),
        ],
        out_specs=pl.BlockSpec((S, H, Dh), lambda b: (b, 0, 0)),
        out_shape=jax.ShapeDtypeStruct((BS, H, Dh), jnp.float32),
        compiler_params=pltpu.CompilerParams(vmem_limit_bytes=VMEM_LIMIT),
    )(qbig, kbig, v)

    out2d = pl.pallas_call(
        _out_proj_body,
        out_shape=jax.ShapeDtypeStruct((BS, D), jnp.float32),
        in_specs=[vmem] * 2,
        out_specs=vmem,
        compiler_params=pltpu.CompilerParams(vmem_limit_bytes=VMEM_LIMIT),
    )(o.reshape(BS, D), Wo)

    return out2d.reshape(B, S, D)


# device time: 106279 ns/iter; 1.5947x vs baseline; 1.5947x over previous
import jax
import jax.numpy as jnp
from jax import lax
from jax.experimental import pallas as pl
from jax.experimental.pallas import tpu as pltpu

B, S, D = 2, 512, 2048
H, Dh, Dr = 16, 128, 32
DC = 128
BS = B * S
SCALE = (Dh + Dr) ** -0.5
VMEM_LIMIT = 60 * 1024 * 1024

NJ = 4
DJ = D // NJ
RJ = (H * Dr) // NJ
NI = 2
SI = S // NI


def _exchange_body(x_ref, wdkv_ref, wuk_ref, wuv_ref, wkr_ref,
                   wq_ref, wqr_ref,
                   q_ref, qr_ref, kr_ref, cfull_ref, wukf_ref, wuvf_ref,
                   c_send, c_recv, wuk_recv, wuv_recv,
                   send_sems, recv_sems):
    j = pl.program_id(0)
    my_x = lax.axis_index("x")
    my_y = lax.axis_index("y")
    my_z = lax.axis_index("z")
    peer = (my_x, my_y, 1 - my_z)

    def _rdma(src, dst, idx):
        return pltpu.make_async_remote_copy(
            src_ref=src, dst_ref=dst,
            send_sem=send_sems.at[idx], recv_sem=recv_sems.at[idx],
            device_id=peer, device_id_type=pl.DeviceIdType.MESH)

    @pl.when(j == 0)
    def _start():
        barrier_sem = pltpu.get_barrier_semaphore()
        pl.semaphore_signal(barrier_sem, inc=1, device_id=peer,
                            device_id_type=pl.DeviceIdType.MESH)
        pl.semaphore_wait(barrier_sem, 1)
        _rdma(wuk_ref, wuk_recv, 0).start()
        _rdma(wuv_ref, wuv_recv, 1).start()
        c_send[...] = jnp.dot(x_ref[...], wdkv_ref[...],
                              preferred_element_type=jnp.float32)
        _rdma(c_send, c_recv, 2).start()
        kr_ref[...] = jnp.dot(x_ref[...], wkr_ref[...],
                              preferred_element_type=jnp.float32)

    q_ref[...] = jnp.dot(x_ref[...], wq_ref[...],
                         preferred_element_type=jnp.float32)
    qr_ref[...] = jnp.dot(x_ref[...], wqr_ref[...],
                          preferred_element_type=jnp.float32)

    @pl.when(j == NJ - 1)
    def _finish():
        _rdma(wuk_ref, wuk_recv, 0).wait()
        _rdma(wuv_ref, wuv_recv, 1).wait()
        _rdma(c_send, c_recv, 2).wait()
        cfull_ref[:, pl.ds(my_z * DC, DC)] = c_send[...]
        cfull_ref[:, pl.ds((1 - my_z) * DC, DC)] = c_recv[...]
        wukf_ref[pl.ds(my_z * DC, DC), :] = wuk_ref[...]
        wukf_ref[pl.ds((1 - my_z) * DC, DC), :] = wuk_recv[...]
        wuvf_ref[pl.ds(my_z * DC, DC), :] = wuv_ref[...]
        wuvf_ref[pl.ds((1 - my_z) * DC, DC), :] = wuv_recv[...]


def _attn_body(q_ref, qr_ref, kr_ref, cfull_ref, wukf_ref, wuvf_ref,
               wo_ref, out_ref, k_scr, v_scr):
    i = pl.program_id(1)

    @pl.when(i == 0)
    def _build_kv():
        k_scr[...] = jnp.dot(cfull_ref[...], wukf_ref[...],
                             preferred_element_type=jnp.float32)
        v_scr[...] = jnp.dot(cfull_ref[...], wuvf_ref[...],
                             preferred_element_type=jnp.float32)

    kr = kr_ref[...]
    acc = None
    for h in range(H):
        q = q_ref[:, h * Dh:(h + 1) * Dh]
        qr = qr_ref[:, h * Dr:(h + 1) * Dr]
        k = k_scr[:, h * Dh:(h + 1) * Dh]
        v = v_scr[:, h * Dh:(h + 1) * Dh]
        scores = (lax.dot_general(q, k, (((1,), (1,)), ((), ())),
                                  preferred_element_type=jnp.float32)
                  + lax.dot_general(qr, kr, (((1,), (1,)), ((), ())),
                                    preferred_element_type=jnp.float32))
        p = jnp.exp(scores * SCALE)
        o = jnp.dot(p, v, preferred_element_type=jnp.float32)
        o = o / jnp.sum(p, axis=1, keepdims=True)
        part = jnp.dot(o, wo_ref[h * Dh:(h + 1) * Dh, :],
                       preferred_element_type=jnp.float32)
        acc = part if acc is None else acc + part
    out_ref[...] = acc


def kernel(x, Wdkv, Wuk, Wuv, Wq, Wqr, Wkr, Wo):
    x2d = x.reshape(BS, D)

    vmem = pl.BlockSpec(memory_space=pltpu.VMEM)
    f32 = jnp.float32

    q2d, qr2d, kr2d, cfull, wukf, wuvf = pl.pallas_call(
        _exchange_body,
        grid=(NJ,),
        out_shape=(
            jax.ShapeDtypeStruct((BS, D), f32),
            jax.ShapeDtypeStruct((BS, H * Dr), f32),
            jax.ShapeDtypeStruct((BS, Dr), f32),
            jax.ShapeDtypeStruct((BS, 2 * DC), f32),
            jax.ShapeDtypeStruct((2 * DC, D), f32),
            jax.ShapeDtypeStruct((2 * DC, D), f32),
        ),
        in_specs=[
            vmem,
            vmem, vmem, vmem, vmem,
            pl.BlockSpec((D, DJ), lambda j: (0, j)),
            pl.BlockSpec((D, RJ), lambda j: (0, j)),
        ],
        out_specs=(
            pl.BlockSpec((BS, DJ), lambda j: (0, j)),
            pl.BlockSpec((BS, RJ), lambda j: (0, j)),
            pl.BlockSpec((BS, Dr), lambda j: (0, 0)),
            pl.BlockSpec((BS, 2 * DC), lambda j: (0, 0)),
            pl.BlockSpec((2 * DC, D), lambda j: (0, 0)),
            pl.BlockSpec((2 * DC, D), lambda j: (0, 0)),
        ),
        scratch_shapes=[
            pltpu.VMEM((BS, DC), f32),
            pltpu.VMEM((BS, DC), f32),
            pltpu.VMEM((DC, D), f32),
            pltpu.VMEM((DC, D), f32),
            pltpu.SemaphoreType.DMA((3,)),
            pltpu.SemaphoreType.DMA((3,)),
        ],
        compiler_params=pltpu.CompilerParams(collective_id=0,
                                             vmem_limit_bytes=VMEM_LIMIT),
    )(x2d, Wdkv, Wuk, Wuv, Wkr, Wq, Wqr)

    out2d = pl.pallas_call(
        _attn_body,
        grid=(B, NI),
        in_specs=[
            pl.BlockSpec((SI, D), lambda b, i: (NI * b + i, 0)),
            pl.BlockSpec((SI, H * Dr), lambda b, i: (NI * b + i, 0)),
            pl.BlockSpec((S, Dr), lambda b, i: (b, 0)),
            pl.BlockSpec((S, 2 * DC), lambda b, i: (b, 0)),
            vmem, vmem,
            vmem,
        ],
        out_specs=pl.BlockSpec((SI, D), lambda b, i: (NI * b + i, 0)),
        out_shape=jax.ShapeDtypeStruct((BS, D), f32),
        scratch_shapes=[
            pltpu.VMEM((S, D), f32),
            pltpu.VMEM((S, D), f32),
        ],
        compiler_params=pltpu.CompilerParams(vmem_limit_bytes=VMEM_LIMIT),
    )(q2d, qr2d, kr2d, cfull, wukf, wuvf, Wo)

    return out2d.reshape(B, S, D)


# device time: 75225 ns/iter; 2.2531x vs baseline; 1.4128x over previous
import jax
import jax.numpy as jnp
from jax import lax
from jax.experimental import pallas as pl
from jax.experimental.pallas import tpu as pltpu

B, S, D = 2, 512, 2048
H, Dh, Dr = 16, 128, 32
DC = 128
BS = B * S
DP = 256
SCALE = (Dh + Dr) ** -0.5
VMEM_LIMIT = 60 * 1024 * 1024

NJ = 4
DJ = D // NJ
HJ = H // NJ
RJ = HJ * Dr
NI = 2
SI = S // NI


def _exchange_body(x_ref, wdkv_ref, wuk_ref, wuv_ref, wkr_ref,
                   wq_ref, wqr_ref,
                   q_ref, kr_ref, cfull_ref, wukf_ref, wuvf_ref,
                   c_send, c_recv, wuk_send, wuk_recv, wuv_send, wuv_recv,
                   send_sems, recv_sems):
    j = pl.program_id(0)
    my_x = lax.axis_index("x")
    my_y = lax.axis_index("y")
    my_z = lax.axis_index("z")
    peer = (my_x, my_y, 1 - my_z)
    bf16 = jnp.bfloat16

    def _rdma(src, dst, idx):
        return pltpu.make_async_remote_copy(
            src_ref=src, dst_ref=dst,
            send_sem=send_sems.at[idx], recv_sem=recv_sems.at[idx],
            device_id=peer, device_id_type=pl.DeviceIdType.MESH)

    @pl.when(j == 0)
    def _start():
        barrier_sem = pltpu.get_barrier_semaphore()
        pl.semaphore_signal(barrier_sem, inc=1, device_id=peer,
                            device_id_type=pl.DeviceIdType.MESH)
        pl.semaphore_wait(barrier_sem, 1)
        wuk_send[...] = wuk_ref[...].astype(bf16)
        wuv_send[...] = wuv_ref[...].astype(bf16)
        _rdma(wuk_send, wuk_recv, 0).start()
        _rdma(wuv_send, wuv_recv, 1).start()
        c_send[...] = jnp.dot(x_ref[...], wdkv_ref[...],
                              preferred_element_type=jnp.float32).astype(bf16)
        _rdma(c_send, c_recv, 2).start()
        kr_ref[...] = jnp.dot(x_ref[...], wkr_ref[...],
                              preferred_element_type=jnp.float32).astype(bf16)

    q2d = jnp.dot(x_ref[...], wq_ref[...],
                  preferred_element_type=jnp.float32)
    qr2d = jnp.dot(x_ref[...], wqr_ref[...],
                   preferred_element_type=jnp.float32)
    q_ref[...] = jnp.zeros((BS, HJ * DP), bf16)
    for hh in range(HJ):
        q_ref[:, hh * DP:hh * DP + Dh] = (
            q2d[:, hh * Dh:(hh + 1) * Dh].astype(bf16))
        q_ref[:, hh * DP + Dh:hh * DP + Dh + Dr] = (
            qr2d[:, hh * Dr:(hh + 1) * Dr].astype(bf16))

    @pl.when(j == NJ - 1)
    def _finish():
        _rdma(wuk_send, wuk_recv, 0).wait()
        _rdma(wuv_send, wuv_recv, 1).wait()
        _rdma(c_send, c_recv, 2).wait()
        cfull_ref[:, pl.ds(my_z * DC, DC)] = c_send[...]
        cfull_ref[:, pl.ds((1 - my_z) * DC, DC)] = c_recv[...]
        wukf_ref[pl.ds(my_z * DC, DC), :] = wuk_send[...]
        wukf_ref[pl.ds((1 - my_z) * DC, DC), :] = wuk_recv[...]
        wuvf_ref[pl.ds(my_z * DC, DC), :] = wuv_send[...]
        wuvf_ref[pl.ds((1 - my_z) * DC, DC), :] = wuv_recv[...]


def _attn_body(q_ref, kr_ref, cfull_ref, wukf_ref, wuvf_ref,
               wo_ref, out_ref, kbig_scr, v_scr, o_scr):
    i = pl.program_id(1)
    bf16 = jnp.bfloat16

    @pl.when(i == 0)
    def _build_kv():
        k2d = jnp.dot(cfull_ref[...], wukf_ref[...],
                      preferred_element_type=jnp.float32)
        v_scr[...] = jnp.dot(cfull_ref[...], wuvf_ref[...],
                             preferred_element_type=jnp.float32).astype(bf16)
        kbig_scr[...] = jnp.zeros((S, H * DP), bf16)
        kr = kr_ref[...]
        for h in range(H):
            kbig_scr[:, h * DP:h * DP + Dh] = (
                k2d[:, h * Dh:(h + 1) * Dh].astype(bf16))
            kbig_scr[:, h * DP + Dh:h * DP + Dh + Dr] = kr

    for h in range(H):
        q = q_ref[:, h * DP:(h + 1) * DP]
        k = kbig_scr[:, h * DP:(h + 1) * DP]
        v = v_scr[:, h * Dh:(h + 1) * Dh]
        scores = lax.dot_general(q, k, (((1,), (1,)), ((), ())),
                                 preferred_element_type=jnp.float32)
        p = jnp.exp(scores * SCALE)
        o = jnp.dot(p.astype(bf16), v, preferred_element_type=jnp.float32)
        o_scr[:, h * Dh:(h + 1) * Dh] = o / jnp.sum(p, axis=1, keepdims=True)

    out_ref[...] = jnp.dot(o_scr[...], wo_ref[...],
                           preferred_element_type=jnp.float32)


def kernel(x, Wdkv, Wuk, Wuv, Wq, Wqr, Wkr, Wo):
    x2d = x.reshape(BS, D)

    vmem = pl.BlockSpec(memory_space=pltpu.VMEM)
    f32 = jnp.float32
    bf16 = jnp.bfloat16

    qbig, kr2d, cfull, wukf, wuvf = pl.pallas_call(
        _exchange_body,
        grid=(NJ,),
        out_shape=(
            jax.ShapeDtypeStruct((BS, H * DP), bf16),
            jax.ShapeDtypeStruct((BS, Dr), bf16),
            jax.ShapeDtypeStruct((BS, 2 * DC), bf16),
            jax.ShapeDtypeStruct((2 * DC, D), bf16),
            jax.ShapeDtypeStruct((2 * DC, D), bf16),
        ),
        in_specs=[
            vmem,
            vmem, vmem, vmem, vmem,
            pl.BlockSpec((D, DJ), lambda j: (0, j)),
            pl.BlockSpec((D, RJ), lambda j: (0, j)),
        ],
        out_specs=(
            pl.BlockSpec((BS, HJ * DP), lambda j: (0, j)),
            pl.BlockSpec((BS, Dr), lambda j: (0, 0)),
            pl.BlockSpec((BS, 2 * DC), lambda j: (0, 0)),
            pl.BlockSpec((2 * DC, D), lambda j: (0, 0)),
            pl.BlockSpec((2 * DC, D), lambda j: (0, 0)),
        ),
        scratch_shapes=[
            pltpu.VMEM((BS, DC), bf16),
            pltpu.VMEM((BS, DC), bf16),
            pltpu.VMEM((DC, D), bf16),
            pltpu.VMEM((DC, D), bf16),
            pltpu.VMEM((DC, D), bf16),
            pltpu.VMEM((DC, D), bf16),
            pltpu.SemaphoreType.DMA((3,)),
            pltpu.SemaphoreType.DMA((3,)),
        ],
        compiler_params=pltpu.CompilerParams(collective_id=0,
                                             vmem_limit_bytes=VMEM_LIMIT),
    )(x2d, Wdkv, Wuk, Wuv, Wkr, Wq, Wqr)

    out2d = pl.pallas_call(
        _attn_body,
        grid=(B, NI),
        in_specs=[
            pl.BlockSpec((SI, H * DP), lambda b, i: (NI * b + i, 0)),
            pl.BlockSpec((S, Dr), lambda b, i: (b, 0)),
            pl.BlockSpec((S, 2 * DC), lambda b, i: (b, 0)),
            vmem, vmem,
            vmem,
        ],
        out_specs=pl.BlockSpec((SI, D), lambda b, i: (NI * b + i, 0)),
        out_shape=jax.ShapeDtypeStruct((BS, D), f32),
        scratch_shapes=[
            pltpu.VMEM((S, H * DP), bf16),
            pltpu.VMEM((S, D), bf16),
            pltpu.VMEM((SI, D), f32),
        ],
        compiler_params=pltpu.CompilerParams(vmem_limit_bytes=VMEM_LIMIT),
    )(qbig, kr2d, cfull, wukf, wuvf, Wo)

    return out2d.reshape(B, S, D)
